# Initial kernel scaffold; baseline (speedup 1.0000x reference)
import jax
import jax.numpy as jnp
from jax.experimental import pallas as pl


def kernel(x, p, edge_index, edge_attr, batch, W0, b0, W1, b1, W2, b2, W3, b3, W4, b4, Wp, bp):
    raise NotImplementedError("write your pallas kernel here")


# speedup vs baseline: 5.1101x; 5.1101x over previous
import functools

import jax
import jax.numpy as jnp
from jax import lax
from jax.experimental import pallas as pl
from jax.experimental.pallas import tpu as pltpu
from jax.experimental.pallas import tpu_sc as plsc

N = 50000
E = 800000
K = 64
G = 256
XD = 5
PD = 3
L = 5

NC = 2
NS = 16
HALF = N // NC
ROWS = 25600
TRASH = HALF
CH = 128
NCH = E // CH
RB = 2000
NRB = N // RB

_mesh = plsc.VectorSubcoreMesh(core_axis_name="c", subcore_axis_name="s")


def _zero_vmem(buf, rows):
    zero = jnp.zeros((16,), jnp.float32)

    def body(r, _):
        for k in range(4):
            buf[r, pl.ds(k * 16, 16)] = zero
        return 0

    lax.fori_loop(0, rows, body, 0)


def _local_dst(didx, lidx, base, j):
    iota = lax.iota(jnp.int32, 16)
    for k in range(8):
        v = didx[pl.ds(k * 16, 16)]
        ld = v - base
        owned = (ld >= 0) & (ld < HALF)
        tr = TRASH + ((iota + k * 16 + (j % 4) * 128) & 511)
        lidx[pl.ds(k * 16, 16)] = jnp.where(owned, ld, tr)


def _prop_body(sims, srcl, dstl, out, hacc, zbuf,
               sidx0, sidx1, didx0, didx1, lidx0, lidx1,
               rows0, rows1, sem0, sem1):
    c = lax.axis_index("c")
    s = lax.axis_index("s")
    base = c * HALF
    sidx = (sidx0, sidx1)
    didx = (didx0, didx1)
    lidx = (lidx0, lidx1)
    rows = (rows0, rows1)
    sems = (sem0, sem1)

    _zero_vmem(zbuf, 100)
    for k in range(16):
        pltpu.sync_copy(zbuf, hacc.at[pl.ds(s * 1600 + k * 100, 100)])
    plsc.subcore_barrier()

    js = (NCH - s + NS - 1) // NS

    @pl.when(js > 0)
    def _prologue():
        ci = s
        pltpu.sync_copy(srcl.at[pl.ds(ci * CH, CH)], sidx[0])
        pltpu.async_copy(sims.at[sidx[0]], rows[0], sems[0])
        pltpu.sync_copy(dstl.at[pl.ds(ci * CH, CH)], didx[0])

    def pair_body(jj, _):
        for par in (0, 1):
            nxt = 1 - par
            j = jj * 2 + par

            @pl.when(j < js)
            def _():
                @pl.when(j + 1 < js)
                def _():
                    ci = s + NS * (j + 1)
                    pltpu.sync_copy(srcl.at[pl.ds(ci * CH, CH)], sidx[nxt])
                    pltpu.async_copy(sims.at[sidx[nxt]], rows[nxt], sems[nxt])
                    pltpu.sync_copy(dstl.at[pl.ds(ci * CH, CH)], didx[nxt])

                pltpu.make_async_copy(sims.at[sidx[par]], rows[par], sems[par]).wait()
                _local_dst(didx[par], lidx[par], base, j)
                pltpu.sync_copy(rows[par], hacc.at[lidx[par]], add=True)

        return 0

    lax.fori_loop(0, (js + 1) // 2, pair_body, 0)
    plsc.subcore_barrier()

    def out_body(jj, _):
        m = s + NS * jj
        pltpu.sync_copy(hacc.at[pl.ds(m * 100, 100)], zbuf)
        pltpu.sync_copy(zbuf, out.at[pl.ds(c * HALF + m * 100, 100)])
        return 0

    lax.fori_loop(0, (250 - s + NS - 1) // NS, out_body, 0)


_prop = functools.partial(
    pl.kernel, _prop_body, mesh=_mesh,
    compiler_params=pltpu.CompilerParams(use_tc_tiling_on_sc=False),
    out_type=jax.ShapeDtypeStruct((N, K), jnp.float32),
    scratch_types=[
        pltpu.VMEM_SHARED((ROWS, K), jnp.float32),
        pltpu.VMEM((100, K), jnp.float32),
        pltpu.VMEM((CH,), jnp.int32), pltpu.VMEM((CH,), jnp.int32),
        pltpu.VMEM((CH,), jnp.int32), pltpu.VMEM((CH,), jnp.int32),
        pltpu.VMEM((CH,), jnp.int32), pltpu.VMEM((CH,), jnp.int32),
        pltpu.VMEM((CH, K), jnp.float32), pltpu.VMEM((CH, K), jnp.float32),
        pltpu.SemaphoreType.DMA, pltpu.SemaphoreType.DMA,
    ],
)()


def _agg_body(dstl, vals, out, aacc, zbuf, didx, lidx, vbuf):
    c = lax.axis_index("c")
    s = lax.axis_index("s")
    base = c * HALF

    zero = jnp.zeros((16,), jnp.float32)

    def zb(i, _):
        zbuf[pl.ds(i * 16, 16)] = zero
        return 0

    lax.fori_loop(0, 100, zb, 0)
    pltpu.sync_copy(zbuf, aacc.at[pl.ds(s * 1600, 1600)])
    plsc.subcore_barrier()

    js = (NCH - s + NS - 1) // NS

    def body(j, _):
        ci = s + NS * j
        pltpu.sync_copy(dstl.at[pl.ds(ci * CH, CH)], didx)
        pltpu.sync_copy(vals.at[pl.ds(ci * CH, CH)], vbuf)
        _local_dst(didx, lidx, base, j)
        pltpu.sync_copy(vbuf, aacc.at[lidx], add=True)
        return 0

    lax.fori_loop(0, js, body, 0)
    plsc.subcore_barrier()

    def out_body(jj, _):
        m = s + NS * jj
        ob = zbuf.at[pl.ds(0, 200)]
        pltpu.sync_copy(aacc.at[pl.ds(m * 200, 200)], ob)
        pltpu.sync_copy(ob, out.at[pl.ds(c * HALF + m * 200, 200)])
        return 0

    lax.fori_loop(0, (125 - s + NS - 1) // NS, out_body, 0)


_agg = functools.partial(
    pl.kernel, _agg_body, mesh=_mesh,
    compiler_params=pltpu.CompilerParams(use_tc_tiling_on_sc=False),
    out_type=jax.ShapeDtypeStruct((N,), jnp.float32),
    scratch_types=[
        pltpu.VMEM_SHARED((ROWS,), jnp.float32),
        pltpu.VMEM((1600,), jnp.float32),
        pltpu.VMEM((CH,), jnp.int32), pltpu.VMEM((CH,), jnp.int32),
        pltpu.VMEM((CH,), jnp.float32),
    ],
)()


def _mm_body(h_ref, p_ref, a_ref, wh_ref, wp_ref, we_ref, b_ref, o_ref):
    acc = jnp.dot(h_ref[:], wh_ref[:], preferred_element_type=jnp.float32)
    acc += jnp.dot(p_ref[:], wp_ref[:], preferred_element_type=jnp.float32)
    acc += jnp.dot(a_ref[:], we_ref[:], preferred_element_type=jnp.float32)
    o_ref[:] = jnp.tanh(acc + b_ref[:])


def _mm(h, p, agg, wh, wp3, we1, b2):
    hd = h.shape[1]
    return pl.pallas_call(
        _mm_body,
        grid=(NRB,),
        in_specs=[
            pl.BlockSpec((RB, hd), lambda i: (i, 0)),
            pl.BlockSpec((RB, PD), lambda i: (i, 0)),
            pl.BlockSpec((RB, 1), lambda i: (i, 0)),
            pl.BlockSpec((hd, K), lambda i: (0, 0)),
            pl.BlockSpec((PD, K), lambda i: (0, 0)),
            pl.BlockSpec((1, K), lambda i: (0, 0)),
            pl.BlockSpec((1, K), lambda i: (0, 0)),
        ],
        out_specs=pl.BlockSpec((RB, K), lambda i: (i, 0)),
        out_shape=jax.ShapeDtypeStruct((N, K), jnp.float32),
    )(h, p, agg, wh, wp3, we1, b2)


def _pool_body(h_ref, b_ref, wp_ref, bp_ref, pred_ref, gr_ref, acc, cnt):
    pid = pl.program_id(0)

    @pl.when(pid == 0)
    def _():
        acc[:] = jnp.zeros_like(acc)
        cnt[:] = jnp.zeros_like(cnt)

    ids = b_ref[0, 0, :]
    gid = lax.broadcasted_iota(jnp.int32, (G, RB), 0)
    onehot = jnp.where(gid == ids[None, :], 1.0, 0.0).astype(jnp.float32)
    acc[:] += jnp.dot(onehot, h_ref[:], preferred_element_type=jnp.float32)
    cnt[:] += jnp.sum(onehot, axis=1, keepdims=True)

    @pl.when(pid == NRB - 1)
    def _():
        gr = acc[:] / jnp.maximum(cnt[:], 1.0)
        gr_ref[:] = gr
        pred_ref[:] = jnp.dot(gr, wp_ref[:],
                              preferred_element_type=jnp.float32) + bp_ref[:]


def _pool(h, batch3, wp, bp2):
    return pl.pallas_call(
        _pool_body,
        grid=(NRB,),
        in_specs=[
            pl.BlockSpec((RB, K), lambda i: (i, 0)),
            pl.BlockSpec((1, 1, RB), lambda i: (i, 0, 0)),
            pl.BlockSpec((K, 1), lambda i: (0, 0)),
            pl.BlockSpec((1, 1), lambda i: (0, 0)),
        ],
        out_specs=[
            pl.BlockSpec((G, 1), lambda i: (0, 0)),
            pl.BlockSpec((G, K), lambda i: (0, 0)),
        ],
        out_shape=[
            jax.ShapeDtypeStruct((G, 1), jnp.float32),
            jax.ShapeDtypeStruct((G, K), jnp.float32),
        ],
        scratch_shapes=[
            pltpu.VMEM((G, K), jnp.float32),
            pltpu.VMEM((G, 1), jnp.float32),
        ],
        compiler_params=pltpu.CompilerParams(
            dimension_semantics=("arbitrary",)),
    )(h, batch3, wp, bp2)


def kernel(x, p, edge_index, edge_attr, batch,
           W0, b0, W1, b1, W2, b2, W3, b3, W4, b4, Wp, bp):
    src = edge_index[0]
    dst = edge_index[1]
    ea = edge_attr[:, 0]

    agg = _agg(dst, ea)
    agg2 = agg.reshape(N, 1)

    h = x
    for (W, b) in ((W0, b0), (W1, b1), (W2, b2), (W3, b3), (W4, b4)):
        hd = W.shape[0] - PD - 1
        sim = _mm(h, p, agg2, W[:hd], W[hd:hd + PD], W[hd + PD:], b.reshape(1, K))
        h = _prop(sim, src, dst)

    pred, gr = _pool(h, batch.reshape(NRB, 1, RB), Wp, bp.reshape(1, 1))
    return (pred, gr)


# speedup vs baseline: 7.3111x; 1.4307x over previous
import functools

import jax
import jax.numpy as jnp
from jax import lax
from jax.experimental import pallas as pl
from jax.experimental.pallas import tpu as pltpu
from jax.experimental.pallas import tpu_sc as plsc

N = 50000
E = 800000
K = 64
G = 256
XD = 5
PD = 3
L = 5

NC = 2
NS = 16
HALF = N // NC
ROWS = 25600
TRASH = HALF
CH = 128
BLK = 1024
CAP = 51200
SZ = NC * NS * CAP
EPT = E // NS
SCB = 2000
RB = 2000
NRB = N // RB

_mesh = plsc.VectorSubcoreMesh(core_axis_name="c", subcore_axis_name="s")
_sc_params = pltpu.CompilerParams(use_tc_tiling_on_sc=False,
                                  needs_layout_passes=False)


def _pre_body(srcl, dstl, vals, agg, slist, dlist, counts,
              aacc, psrc, pdst, sblk, dblk, vblk, lidx, zbuf, cbuf):
    c = lax.axis_index("c")
    s = lax.axis_index("s")
    base = c * HALF
    iota = lax.iota(jnp.int32, 16)
    zero = jnp.zeros((16,), jnp.float32)

    def zb(i, _):
        zbuf[pl.ds(i * 16, 16)] = zero
        return 0

    lax.fori_loop(0, 100, zb, 0)
    pltpu.sync_copy(zbuf, aacc.at[pl.ds(s * 1600, 1600)])
    plsc.subcore_barrier()

    def blk_body(b, ptr):
        off = s * EPT + b * SCB
        pltpu.sync_copy(srcl.at[pl.ds(off, SCB)], sblk)
        pltpu.sync_copy(dstl.at[pl.ds(off, SCB)], dblk)
        pltpu.sync_copy(vals.at[pl.ds(off, SCB)], vblk)

        def sub_body(q, ptr):
            for g in range(5):
                i0 = q * 80 + g * 16
                d = dblk[pl.ds(i0, 16)]
                sv = sblk[pl.ds(i0, 16)]
                ld = d - base
                owned = (ld >= 0) & (ld < HALF)
                lidx[pl.ds(g * 16, 16)] = jnp.where(
                    owned, ld, TRASH + ((iota + i0) & 511))
                psrc[pl.ds(ptr, 16)] = sv
                pdst[pl.ds(ptr, 16)] = jnp.where(
                    owned, ld, TRASH + ((iota + i0) & 511))
                ptr = ptr + 16
            pltpu.sync_copy(vblk.at[pl.ds(q * 80, 80)], aacc.at[lidx],
                            add=True)
            return ptr

        return lax.fori_loop(0, SCB // 80, sub_body, ptr)

    ptr = lax.fori_loop(0, EPT // SCB, blk_body, 0)

    npad = ((BLK - ptr % BLK) % BLK) // 16

    def pad_body(g, ptr):
        psrc[pl.ds(ptr, 16)] = (iota + g * 16) & 4095
        pdst[pl.ds(ptr, 16)] = TRASH + ((iota + g * 16) & 511)
        return ptr + 16

    ptr = lax.fori_loop(0, npad, pad_body, ptr)
    nblk = ptr // BLK

    lbase = (c * NS + s) * CAP

    def wr(b, _):
        pltpu.sync_copy(psrc.at[pl.ds(b * BLK, BLK)],
                        slist.at[pl.ds(lbase + b * BLK, BLK)])
        pltpu.sync_copy(pdst.at[pl.ds(b * BLK, BLK)],
                        dlist.at[pl.ds(lbase + b * BLK, BLK)])
        return 0

    lax.fori_loop(0, nblk, wr, 0)

    cbuf[pl.ds(0, 16)] = jnp.full((16,), nblk, jnp.int32)
    pltpu.sync_copy(cbuf, counts.at[pl.ds((c * NS + s) * 16, 16)])

    plsc.subcore_barrier()

    def out_body(jj, _):
        m = s + NS * jj
        ob = zbuf.at[pl.ds(0, 200)]
        pltpu.sync_copy(aacc.at[pl.ds(m * 200, 200)], ob)
        pltpu.sync_copy(ob, agg.at[pl.ds(c * HALF + m * 200, 200)])
        return 0

    lax.fori_loop(0, (125 - s + NS - 1) // NS, out_body, 0)


_pre = functools.partial(
    pl.kernel, _pre_body, mesh=_mesh, compiler_params=_sc_params,
    out_type=[
        jax.ShapeDtypeStruct((N,), jnp.float32),
        jax.ShapeDtypeStruct((SZ,), jnp.int32),
        jax.ShapeDtypeStruct((SZ,), jnp.int32),
        jax.ShapeDtypeStruct((NC * NS * 16,), jnp.int32),
    ],
    scratch_types=[
        pltpu.VMEM_SHARED((ROWS,), jnp.float32),
        pltpu.VMEM((CAP,), jnp.int32), pltpu.VMEM((CAP,), jnp.int32),
        pltpu.VMEM((SCB,), jnp.int32), pltpu.VMEM((SCB,), jnp.int32),
        pltpu.VMEM((SCB,), jnp.float32),
        pltpu.VMEM((80,), jnp.int32),
        pltpu.VMEM((1600,), jnp.float32),
        pltpu.VMEM((16,), jnp.int32),
    ],
)()


def _prop_body(sims, slist, dlist, counts, out, hacc, zbuf,
               sblk, dblk, dfix, sfix0, sfix1, cbuf, rows0, rows1,
               sem0, sem1):
    c = lax.axis_index("c")
    s = lax.axis_index("s")
    rows = (rows0, rows1)
    sems = (sem0, sem1)
    sfix = (sfix0, sfix1)

    zero = jnp.zeros((16,), jnp.float32)

    def zb(r, _):
        for k in range(4):
            zbuf[r, pl.ds(k * 16, 16)] = zero
        return 0

    lax.fori_loop(0, 100, zb, 0)
    for k in range(16):
        pltpu.sync_copy(zbuf, hacc.at[pl.ds(s * 1600 + k * 100, 100)])
    plsc.subcore_barrier()

    pltpu.sync_copy(counts.at[pl.ds((c * NS + s) * 16, 16)], cbuf)
    nblk = jnp.max(cbuf[pl.ds(0, 16)])
    lbase = (c * NS + s) * CAP

    def _fire(kk, par):
        sf = sfix[par]
        for g in range(8):
            sf[pl.ds(g * 16, 16)] = sblk[pl.ds(kk * CH + g * 16, 16)]
        pltpu.async_copy(sims.at[sf], rows[par], sems[par])

    def blk_body(b, _):
        boff = lbase + b * BLK
        pltpu.sync_copy(slist.at[pl.ds(boff, BLK)], sblk)
        pltpu.sync_copy(dlist.at[pl.ds(boff, BLK)], dblk)
        _fire(0, 0)
        _fire(1, 1)
        for kk in range(8):
            par = kk % 2
            pltpu.make_async_copy(sims.at[sfix[par]],
                                  rows[par], sems[par]).wait()
            for g in range(8):
                dfix[pl.ds(g * 16, 16)] = dblk[pl.ds(kk * CH + g * 16, 16)]
            pltpu.sync_copy(rows[par], hacc.at[dfix], add=True)
            if kk + 2 < 8:
                _fire(kk + 2, par)
        return 0

    lax.fori_loop(0, nblk, blk_body, 0)
    plsc.subcore_barrier()

    def out_body(jj, _):
        m = s + NS * jj
        pltpu.sync_copy(hacc.at[pl.ds(m * 100, 100)], zbuf)
        pltpu.sync_copy(zbuf, out.at[pl.ds(c * HALF + m * 100, 100)])
        return 0

    lax.fori_loop(0, (250 - s + NS - 1) // NS, out_body, 0)


_prop = functools.partial(
    pl.kernel, _prop_body, mesh=_mesh, compiler_params=_sc_params,
    out_type=jax.ShapeDtypeStruct((N, K), jnp.float32),
    scratch_types=[
        pltpu.VMEM_SHARED((ROWS, K), jnp.float32),
        pltpu.VMEM((100, K), jnp.float32),
        pltpu.VMEM((BLK,), jnp.int32), pltpu.VMEM((BLK,), jnp.int32),
        pltpu.VMEM((CH,), jnp.int32),
        pltpu.VMEM((CH,), jnp.int32), pltpu.VMEM((CH,), jnp.int32),
        pltpu.VMEM((16,), jnp.int32),
        pltpu.VMEM((CH, K), jnp.float32), pltpu.VMEM((CH, K), jnp.float32),
        pltpu.SemaphoreType.DMA, pltpu.SemaphoreType.DMA,
    ],
)()


def _mm_body(h_ref, p_ref, a_ref, wh_ref, wp_ref, we_ref, b_ref, o_ref):
    acc = jnp.dot(h_ref[:], wh_ref[:], preferred_element_type=jnp.float32)
    acc += jnp.dot(p_ref[:], wp_ref[:], preferred_element_type=jnp.float32)
    acc += jnp.dot(a_ref[:], we_ref[:], preferred_element_type=jnp.float32)
    o_ref[:] = jnp.tanh(acc + b_ref[:])


def _mm(h, p, agg, wh, wp3, we1, b2):
    hd = h.shape[1]
    return pl.pallas_call(
        _mm_body,
        grid=(NRB,),
        in_specs=[
            pl.BlockSpec((RB, hd), lambda i: (i, 0)),
            pl.BlockSpec((RB, PD), lambda i: (i, 0)),
            pl.BlockSpec((RB, 1), lambda i: (i, 0)),
            pl.BlockSpec((hd, K), lambda i: (0, 0)),
            pl.BlockSpec((PD, K), lambda i: (0, 0)),
            pl.BlockSpec((1, K), lambda i: (0, 0)),
            pl.BlockSpec((1, K), lambda i: (0, 0)),
        ],
        out_specs=pl.BlockSpec((RB, K), lambda i: (i, 0)),
        out_shape=jax.ShapeDtypeStruct((N, K), jnp.float32),
    )(h, p, agg, wh, wp3, we1, b2)


def _pool_body(h_ref, b_ref, wp_ref, bp_ref, pred_ref, gr_ref, acc, cnt):
    pid = pl.program_id(0)

    @pl.when(pid == 0)
    def _():
        acc[:] = jnp.zeros_like(acc)
        cnt[:] = jnp.zeros_like(cnt)

    ids = b_ref[0, 0, :]
    gid = lax.broadcasted_iota(jnp.int32, (G, RB), 0)
    onehot = jnp.where(gid == ids[None, :], 1.0, 0.0).astype(jnp.float32)
    acc[:] += jnp.dot(onehot, h_ref[:], preferred_element_type=jnp.float32)
    cnt[:] += jnp.sum(onehot, axis=1, keepdims=True)

    @pl.when(pid == NRB - 1)
    def _():
        gr = acc[:] / jnp.maximum(cnt[:], 1.0)
        gr_ref[:] = gr
        pred_ref[:] = jnp.dot(gr, wp_ref[:],
                              preferred_element_type=jnp.float32) + bp_ref[:]


def _pool(h, batch3, wp, bp2):
    return pl.pallas_call(
        _pool_body,
        grid=(NRB,),
        in_specs=[
            pl.BlockSpec((RB, K), lambda i: (i, 0)),
            pl.BlockSpec((1, 1, RB), lambda i: (i, 0, 0)),
            pl.BlockSpec((K, 1), lambda i: (0, 0)),
            pl.BlockSpec((1, 1), lambda i: (0, 0)),
        ],
        out_specs=[
            pl.BlockSpec((G, 1), lambda i: (0, 0)),
            pl.BlockSpec((G, K), lambda i: (0, 0)),
        ],
        out_shape=[
            jax.ShapeDtypeStruct((G, 1), jnp.float32),
            jax.ShapeDtypeStruct((G, K), jnp.float32),
        ],
        scratch_shapes=[
            pltpu.VMEM((G, K), jnp.float32),
            pltpu.VMEM((G, 1), jnp.float32),
        ],
        compiler_params=pltpu.CompilerParams(
            dimension_semantics=("arbitrary",)),
    )(h, batch3, wp, bp2)


def kernel(x, p, edge_index, edge_attr, batch,
           W0, b0, W1, b1, W2, b2, W3, b3, W4, b4, Wp, bp):
    src = edge_index[0]
    dst = edge_index[1]
    ea = edge_attr[:, 0]

    agg, slist, dlist, counts = _pre(src, dst, ea)
    agg2 = agg.reshape(N, 1)

    h = x
    for (W, b) in ((W0, b0), (W1, b1), (W2, b2), (W3, b3), (W4, b4)):
        hd = W.shape[0] - PD - 1
        sim = _mm(h, p, agg2, W[:hd], W[hd:hd + PD], W[hd + PD:], b.reshape(1, K))
        h = _prop(sim, slist, dlist, counts)

    pred, gr = _pool(h, batch.reshape(NRB, 1, RB), Wp, bp.reshape(1, 1))
    return (pred, gr)
